# baseline (speedup 1.0000x reference)
import functools

import jax
import jax.numpy as jnp
from jax import lax
from jax.experimental import pallas as pl
from jax.experimental.pallas import tpu as pltpu
from jax.experimental.pallas import tpu_sc as plsc

B = 16384
C = 10
NC = 2
NS = 16
NW = NC * NS
RPW = B // NW
CHUNK = 128
NCH = RPW // CHUNK
W8 = (1_000_000 * C) // 8

BETA = 0.7
LAMBDA_ = 3.0
CLIP_LO = 0.0001
CLIP_HI = 1.0 - 0.0001

_sc_mesh = plsc.VectorSubcoreMesh(core_axis_name="c", subcore_axis_name="s")


@functools.partial(
    pl.kernel,
    mesh=_sc_mesh,
    compiler_params=pltpu.CompilerParams(use_tc_tiling_on_sc=False),
    out_type=(
        jax.ShapeDtypeStruct((NW * NCH, CHUNK, 8), jnp.float32),
        jax.ShapeDtypeStruct((NW * NCH, CHUNK, 8), jnp.float32),
    ),
    scratch_types=[
        pltpu.VMEM((NCH, CHUNK), jnp.int32),
        pltpu.VMEM((NCH, CHUNK), jnp.int32),
        pltpu.VMEM((NCH, CHUNK), jnp.int32),
        pltpu.VMEM((NCH, CHUNK, 8), jnp.float32),
        pltpu.VMEM((NCH, CHUNK, 8), jnp.float32),
        pltpu.SemaphoreType.DMA,
    ],
)
def _sc_gather(idx_hbm, tgt_hbm, outa_hbm, outb_hbm,
               idx_v, ja_v, jb_v, bufa_v, bufb_v, sem):
    wid = lax.axis_index("s") * NC + lax.axis_index("c")
    tgt8 = tgt_hbm
    pltpu.sync_copy(idx_hbm.at[pl.ds(wid * NCH, NCH)], idx_v)
    for ch in range(NCH):
        for g in range(CHUNK // 16):
            iv = idx_v[ch, pl.ds(g * 16, 16)]
            ja = (iv * C) >> 3
            ja_v[ch, pl.ds(g * 16, 16)] = ja
            jb_v[ch, pl.ds(g * 16, 16)] = ja + 1
    copies = []
    for ch in range(NCH):
        copies.append(pltpu.async_copy(tgt8.at[ja_v.at[ch]], bufa_v.at[ch], sem))
        copies.append(pltpu.async_copy(tgt8.at[jb_v.at[ch]], bufb_v.at[ch], sem))
    for cp in copies:
        cp.wait()
    pltpu.sync_copy(bufa_v, outa_hbm.at[pl.ds(wid * NCH, NCH)])
    pltpu.sync_copy(bufb_v, outb_hbm.at[pl.ds(wid * NCH, NCH)])


def _tc_loss_body(out_t_ref, lab_ref, idx_ref, a_t_ref, b_t_ref, loss_ref):
    x = out_t_ref[...]
    m = jnp.max(x, axis=0, keepdims=True)
    e = jnp.exp(x - m)
    s = jnp.sum(e, axis=0, keepdims=True)
    y = jnp.clip(e / s, CLIP_LO, CLIP_HI)
    ysum = jnp.sum(y, axis=0, keepdims=True)
    w16 = jnp.concatenate([a_t_ref[...], b_t_ref[...]], axis=0)
    r = (idx_ref[0, :] & 3)[None, :]
    tr = jnp.where(
        r == 0, w16[0:C],
        jnp.where(r == 1, w16[2:C + 2],
                  jnp.where(r == 2, w16[4:C + 4], w16[6:C + 6])))
    nr = BETA * tr + (1.0 - BETA) * (y / ysum)
    dot = jnp.sum(nr * y, axis=0)
    lab = lab_ref[0, :]
    cls = lax.broadcasted_iota(jnp.int32, x.shape, 0)
    xl = jnp.sum(jnp.where(cls == lab[None, :], x, 0.0), axis=0)
    ce = (m[0] + jnp.log(s[0])) - xl
    elr = jnp.log(1.0 - dot)
    loss_ref[0, 0] = jnp.mean(ce) + LAMBDA_ * jnp.mean(elr)


_tc_loss = pl.pallas_call(
    _tc_loss_body,
    out_shape=jax.ShapeDtypeStruct((1, 1), jnp.float32),
    out_specs=pl.BlockSpec(memory_space=pltpu.SMEM),
)


def kernel(index, output, label, target):
    raw_a, raw_b = _sc_gather(index.reshape(NW * NCH, CHUNK),
                              target.reshape(W8, 8))
    loss = _tc_loss(
        output.T,
        label[None, :],
        index[None, :],
        raw_a.reshape(B, 8).T,
        raw_b.reshape(B, 8).T,
    )
    return loss[0, 0]
